# baseline (speedup 1.0000x reference)
import functools

import jax
import jax.numpy as jnp
from jax import lax
from jax.experimental import pallas as pl
from jax.experimental.pallas import tpu as pltpu
from jax.experimental.pallas import tpu_sc as plsc

VOCAB = 1_000_000
DIM = 32
B = 16384
F = 26
BB = B // 128
ITEMS = F * BB
ROWS_TOTAL = B * F

_INFO = plsc.get_sparse_core_info()
NC = _INFO.num_cores
NS = _INFO.num_subcores
NW = NC * NS
PER_W = ITEMS // NW
PITCH = 129


@functools.partial(
    pl.kernel,
    out_type=jax.ShapeDtypeStruct((F, 4, BB, 8, 128), jnp.float32),
    mesh=plsc.VectorSubcoreMesh(core_axis_name="c", subcore_axis_name="s"),
    compiler_params=pltpu.CompilerParams(
        use_tc_tiling_on_sc=False, needs_layout_passes=False
    ),
    scratch_types=[
        pltpu.VMEM((PER_W * 128,), jnp.int32),
        pltpu.VMEM((2, 128, DIM), jnp.float32),
        pltpu.VMEM((2, 4, 8, PITCH), jnp.float32),
        pltpu.SemaphoreType.DMA,
        pltpu.SemaphoreType.DMA,
    ],
)
def _gather_kernel(idx_hbm, table2d, out_hbm, idx_v, rows_v, trows_v, gsem, osem):
    wid = lax.axis_index("s") * NC + lax.axis_index("c")
    t0 = wid * PER_W
    pltpu.sync_copy(idx_hbm.at[pl.ds(t0 * 128, PER_W * 128)], idx_v)

    iota = lax.iota(jnp.int32, 16)
    ds_idx = lax.rem(iota, 8)
    db_lo = lax.div(iota, 8)
    db_hi = db_lo + 2

    def gather(t, buf):
        return pltpu.async_copy(
            table2d.at[idx_v.at[pl.ds(t * 128, 128)]],
            rows_v.at[buf],
            gsem,
        )

    def transpose(buf):
        for b in range(128):
            b_vec = jnp.full((16,), b, jnp.int32)
            for h in range(2):
                v = rows_v[buf, b, pl.ds(16 * h, 16)]
                plsc.store_scatter(
                    trows_v.at[buf],
                    [db_hi if h else db_lo, ds_idx, b_vec],
                    v,
                )

    def writeout(t, buf):
        g = t0 + t
        f = g // BB
        bb = g - f * BB
        src = trows_v.at[buf, :, :, pl.ds(0, 128)]
        return pltpu.async_copy(src, out_hbm.at[f, :, bb], osem)

    def drain_out(t, buf):
        g = t0 + t
        f = g // BB
        bb = g - f * BB
        src = trows_v.at[buf, :, :, pl.ds(0, 128)]
        pltpu.make_async_copy(src, out_hbm.at[f, :, bb], osem).wait()

    def drain_gather(t, buf):
        pltpu.make_async_copy(
            table2d.at[idx_v.at[pl.ds(t * 128, 128)]],
            rows_v.at[buf],
            gsem,
        ).wait()

    gather(0, 0)

    def body(t, carry):
        for buf in range(2):
            tt = t + buf
            drain_gather(tt, buf)
            nxt = tt + 1

            @pl.when(nxt < PER_W)
            def _():
                gather(nxt, 1 - buf)

            @pl.when(tt >= 2)
            def _():
                drain_out(tt - 2, buf)

            transpose(buf)
            writeout(tt, buf)
        return carry

    lax.fori_loop(0, PER_W // 2, lambda i, c: body(2 * i, c), 0, unroll=False)
    drain_out(PER_W - 2, 0)
    drain_out(PER_W - 1, 1)


_VBLK = 512
_NBLK = (VOCAB + _VBLK - 1) // _VBLK


def _detile_body(sel_ref, lt_ref, out_ref):
    xt = lt_ref[...].T
    for q in range(4):
        out_ref[:, DIM * q:DIM * (q + 1)] = jnp.dot(
            sel_ref[q], xt, preferred_element_type=jnp.float32
        )


_detile = pl.pallas_call(
    _detile_body,
    grid=(_NBLK,),
    in_specs=[
        pl.BlockSpec((4, _VBLK // 4, _VBLK), lambda i: (0, 0, 0)),
        pl.BlockSpec((DIM, _VBLK), lambda i: (0, i)),
    ],
    out_specs=pl.BlockSpec((_VBLK // 4, 4 * DIM), lambda i: (i, 0)),
    out_shape=jax.ShapeDtypeStruct((VOCAB // 4, 4 * DIM), jnp.float32),
)


def kernel(inputs, L):
    idx_t = inputs.T.reshape(-1).astype(jnp.int32)
    r_idx = lax.broadcasted_iota(jnp.int32, (4, _VBLK // 4, _VBLK), 1)
    q_idx = lax.broadcasted_iota(jnp.int32, (4, _VBLK // 4, _VBLK), 0)
    j_idx = lax.broadcasted_iota(jnp.int32, (4, _VBLK // 4, _VBLK), 2)
    sel = (j_idx == 4 * r_idx + q_idx).astype(jnp.float32)
    table_lin = _detile(sel, L.T).reshape(VOCAB, DIM)
    p5 = _gather_kernel(idx_t, table_lin)
    return p5.transpose(2, 4, 0, 1, 3).reshape(B, F, DIM)


# speedup vs baseline: 1.0038x; 1.0038x over previous
import functools

import jax
import jax.numpy as jnp
from jax import lax
from jax.experimental import pallas as pl
from jax.experimental.pallas import tpu as pltpu
from jax.experimental.pallas import tpu_sc as plsc

VOCAB = 1_000_000
DIM = 32
B = 16384
F = 26
BB = B // 128
ITEMS = F * BB
ROWS_TOTAL = B * F

_INFO = plsc.get_sparse_core_info()
NC = _INFO.num_cores
NS = _INFO.num_subcores
NW = NC * NS
PER_W = ITEMS // NW
PITCH = 129


@functools.partial(
    pl.kernel,
    out_type=jax.ShapeDtypeStruct((F, 4, BB, 8, 128), jnp.float32),
    mesh=plsc.VectorSubcoreMesh(core_axis_name="c", subcore_axis_name="s"),
    compiler_params=pltpu.CompilerParams(
        use_tc_tiling_on_sc=False, needs_layout_passes=False
    ),
    scratch_types=[
        pltpu.VMEM((PER_W * 128,), jnp.int32),
        pltpu.VMEM((2, 128, DIM), jnp.float32),
        pltpu.VMEM((2, 4, 8, PITCH), jnp.float32),
        pltpu.SemaphoreType.DMA,
        pltpu.SemaphoreType.DMA,
    ],
)
def _gather_kernel(idx_hbm, table2d, out_hbm, idx_v, rows_v, trows_v, gsem, osem):
    wid = lax.axis_index("s") * NC + lax.axis_index("c")
    t0 = wid * PER_W
    pltpu.sync_copy(idx_hbm.at[pl.ds(t0 * 128, PER_W * 128)], idx_v)

    iota = lax.iota(jnp.int32, 16)
    ds_idx = lax.rem(iota, 8)
    db_lo = lax.div(iota, 8)
    db_hi = db_lo + 2

    def gather(t, buf):
        return pltpu.async_copy(
            table2d.at[idx_v.at[pl.ds(t * 128, 128)]],
            rows_v.at[buf],
            gsem,
        )

    def transpose(buf):
        for b in range(128):
            b_vec = jnp.full((16,), b, jnp.int32)
            for h in range(2):
                v = rows_v[buf, b, pl.ds(16 * h, 16)]
                plsc.store_scatter(
                    trows_v.at[buf],
                    [db_hi if h else db_lo, ds_idx, b_vec],
                    v,
                )

    def writeout(t, buf):
        g = t0 + t
        f = g // BB
        bb = g - f * BB
        src = trows_v.at[buf, :, :, pl.ds(0, 128)]
        return pltpu.async_copy(src, out_hbm.at[f, :, bb], osem)

    def drain_out(t, buf):
        g = t0 + t
        f = g // BB
        bb = g - f * BB
        src = trows_v.at[buf, :, :, pl.ds(0, 128)]
        pltpu.make_async_copy(src, out_hbm.at[f, :, bb], osem).wait()

    def drain_gather(t, buf):
        pltpu.make_async_copy(
            table2d.at[idx_v.at[pl.ds(t * 128, 128)]],
            rows_v.at[buf],
            gsem,
        ).wait()

    gather(0, 0)

    def body(t, carry):
        for buf in range(2):
            tt = t + buf
            drain_gather(tt, buf)
            nxt = tt + 1

            @pl.when(nxt < PER_W)
            def _():
                gather(nxt, 1 - buf)

            @pl.when(tt >= 2)
            def _():
                drain_out(tt - 2, buf)

            transpose(buf)
            writeout(tt, buf)
        return carry

    lax.fori_loop(0, PER_W // 2, lambda i, c: body(2 * i, c), 0, unroll=False)
    drain_out(PER_W - 2, 0)
    drain_out(PER_W - 1, 1)


_VBLK = 512
_NBLK = (VOCAB + _VBLK - 1) // _VBLK


def _detile_body(lt_ref, out_ref, sel_ref):
    @pl.when(pl.program_id(0) == 0)
    def _():
        r_idx = lax.broadcasted_iota(jnp.int32, (4, _VBLK // 4, _VBLK), 1)
        q_idx = lax.broadcasted_iota(jnp.int32, (4, _VBLK // 4, _VBLK), 0)
        j_idx = lax.broadcasted_iota(jnp.int32, (4, _VBLK // 4, _VBLK), 2)
        sel_ref[...] = (j_idx == 4 * r_idx + q_idx).astype(jnp.float32)

    for q in range(4):
        out_ref[:, DIM * q:DIM * (q + 1)] = lax.dot_general(
            sel_ref[q],
            lt_ref[...],
            (((1,), (1,)), ((), ())),
            preferred_element_type=jnp.float32,
        )


_detile = pl.pallas_call(
    _detile_body,
    grid=(_NBLK,),
    in_specs=[pl.BlockSpec((DIM, _VBLK), lambda i: (0, i))],
    out_specs=pl.BlockSpec((_VBLK // 4, 4 * DIM), lambda i: (i, 0)),
    out_shape=jax.ShapeDtypeStruct((VOCAB // 4, 4 * DIM), jnp.float32),
    scratch_shapes=[pltpu.VMEM((4, _VBLK // 4, _VBLK), jnp.float32)],
)


def kernel(inputs, L):
    idx_t = inputs.T.reshape(-1).astype(jnp.int32)
    table_lin = _detile(L.T).reshape(VOCAB, DIM)
    p5 = _gather_kernel(idx_t, table_lin)
    return p5.transpose(2, 4, 0, 1, 3).reshape(B, F, DIM)


# speedup vs baseline: 1.0485x; 1.0446x over previous
import functools

import jax
import jax.numpy as jnp
from jax import lax
from jax.experimental import pallas as pl
from jax.experimental.pallas import tpu as pltpu
from jax.experimental.pallas import tpu_sc as plsc

VOCAB = 1_000_000
DIM = 32
B = 16384
F = 26
BB = B // 128
ITEMS = F * BB
ROWS_TOTAL = B * F

_INFO = plsc.get_sparse_core_info()
NC = _INFO.num_cores
NS = _INFO.num_subcores
NW = NC * NS
PER_W = ITEMS // NW
PITCH = 129


@functools.partial(
    pl.kernel,
    out_type=jax.ShapeDtypeStruct((F, 4, BB, 8, 128), jnp.float32),
    mesh=plsc.VectorSubcoreMesh(core_axis_name="c", subcore_axis_name="s"),
    compiler_params=pltpu.CompilerParams(
        use_tc_tiling_on_sc=False, needs_layout_passes=False
    ),
    scratch_types=[
        pltpu.VMEM((PER_W * 128,), jnp.int32),
        pltpu.VMEM((2, 128, DIM), jnp.float32),
        pltpu.VMEM((2, 4, 8, PITCH), jnp.float32),
        pltpu.SemaphoreType.DMA,
        pltpu.SemaphoreType.DMA,
    ],
)
def _gather_kernel(idx_hbm, table2d, out_hbm, idx_v, rows_v, trows_v, gsem, osem):
    wid = lax.axis_index("s") * NC + lax.axis_index("c")
    t0 = wid * PER_W
    pltpu.sync_copy(idx_hbm.at[pl.ds(t0 * 128, PER_W * 128)], idx_v)

    iota = lax.iota(jnp.int32, 16)
    ds_idx = lax.rem(iota, 8)
    db_lo = lax.div(iota, 8)
    db_hi = db_lo + 2

    def gather(t, buf):
        return pltpu.async_copy(
            table2d.at[idx_v.at[pl.ds(t * 128, 128)]],
            rows_v.at[buf],
            gsem,
        )

    def transpose(buf):
        for b in range(128):
            b_vec = jnp.full((16,), b, jnp.int32)
            for h in range(2):
                v = rows_v[buf, b, pl.ds(16 * h, 16)]
                plsc.store_scatter(
                    trows_v.at[buf],
                    [db_hi if h else db_lo, ds_idx, b_vec],
                    v,
                )

    def writeout(t, buf):
        g = t0 + t
        f = g // BB
        bb = g - f * BB
        src = trows_v.at[buf, :, :, pl.ds(0, 128)]
        return pltpu.async_copy(src, out_hbm.at[f, :, bb], osem)

    def drain_out(t, buf):
        g = t0 + t
        f = g // BB
        bb = g - f * BB
        src = trows_v.at[buf, :, :, pl.ds(0, 128)]
        pltpu.make_async_copy(src, out_hbm.at[f, :, bb], osem).wait()

    def drain_gather(t, buf):
        pltpu.make_async_copy(
            table2d.at[idx_v.at[pl.ds(t * 128, 128)]],
            rows_v.at[buf],
            gsem,
        ).wait()

    gather(0, 0)

    def body(t, carry):
        for buf in range(2):
            tt = t + buf
            drain_gather(tt, buf)
            nxt = tt + 1

            @pl.when(nxt < PER_W)
            def _():
                gather(nxt, 1 - buf)

            @pl.when(tt >= 2)
            def _():
                drain_out(tt - 2, buf)

            transpose(buf)
            writeout(tt, buf)
        return carry

    lax.fori_loop(0, PER_W // 2, lambda i, c: body(2 * i, c), 0, unroll=False)
    drain_out(PER_W - 2, 0)
    drain_out(PER_W - 1, 1)


_VBLK = 512
_NBLK = (VOCAB + _VBLK - 1) // _VBLK


def _detile_body(lt_ref, out_ref):
    xt = lt_ref[...].T.reshape(_VBLK // 4, 4, DIM)
    out_ref[...] = jnp.concatenate([xt[:, q, :] for q in range(4)], axis=1)


_detile = pl.pallas_call(
    _detile_body,
    grid=(_NBLK,),
    in_specs=[pl.BlockSpec((DIM, _VBLK), lambda i: (0, i))],
    out_specs=pl.BlockSpec((_VBLK // 4, 4 * DIM), lambda i: (i, 0)),
    out_shape=jax.ShapeDtypeStruct((VOCAB // 4, 4 * DIM), jnp.float32),
)


def kernel(inputs, L):
    idx_t = inputs.T.reshape(-1).astype(jnp.int32)
    table_lin = _detile(L.T).reshape(VOCAB, DIM)
    p5 = _gather_kernel(idx_t, table_lin)
    return p5.transpose(2, 4, 0, 1, 3).reshape(B, F, DIM)


# speedup vs baseline: 2.2518x; 2.1476x over previous
import functools

import jax
import jax.numpy as jnp
from jax import lax
from jax.experimental import pallas as pl
from jax.experimental.pallas import tpu as pltpu
from jax.experimental.pallas import tpu_sc as plsc

VOCAB = 1_000_000
DIM = 32
B = 16384
F = 26
BB = B // 128
ITEMS = F * BB
ROWS_TOTAL = B * F

_INFO = plsc.get_sparse_core_info()
NC = _INFO.num_cores
NS = _INFO.num_subcores
NW = NC * NS
PER_W = ITEMS // NW
PITCH = 129


@functools.partial(
    pl.kernel,
    out_type=jax.ShapeDtypeStruct((F, 4, BB, 8, 128), jnp.float32),
    mesh=plsc.VectorSubcoreMesh(core_axis_name="c", subcore_axis_name="s"),
    compiler_params=pltpu.CompilerParams(
        use_tc_tiling_on_sc=False, needs_layout_passes=False
    ),
    scratch_types=[
        pltpu.VMEM((PER_W * 128,), jnp.int32),
        pltpu.VMEM((2, 128, DIM), jnp.float32),
        pltpu.VMEM((2, 4, 8, PITCH), jnp.float32),
        pltpu.SemaphoreType.DMA,
        pltpu.SemaphoreType.DMA,
    ],
)
def _gather_kernel(idx_hbm, table2d, out_hbm, idx_v, rows_v, trows_v, gsem, osem):
    wid = lax.axis_index("s") * NC + lax.axis_index("c")
    t0 = wid * PER_W
    pltpu.sync_copy(idx_hbm.at[pl.ds(t0 * 128, PER_W * 128)], idx_v)

    iota = lax.iota(jnp.int32, 16)
    ds_idx = lax.rem(iota, 8)
    db_lo = lax.div(iota, 8)
    db_hi = db_lo + 2

    def gather(t, buf):
        return pltpu.async_copy(
            table2d.at[idx_v.at[pl.ds(t * 128, 128)]],
            rows_v.at[buf],
            gsem,
        )

    def transpose(buf):
        for b in range(128):
            b_vec = jnp.full((16,), b, jnp.int32)
            for h in range(2):
                v = rows_v[buf, b, pl.ds(16 * h, 16)]
                plsc.store_scatter(
                    trows_v.at[buf],
                    [db_hi if h else db_lo, ds_idx, b_vec],
                    v,
                )

    def writeout(t, buf):
        g = t0 + t
        f = g // BB
        bb = g - f * BB
        src = trows_v.at[buf, :, :, pl.ds(0, 128)]
        return pltpu.async_copy(src, out_hbm.at[f, :, bb], osem)

    def drain_out(t, buf):
        g = t0 + t
        f = g // BB
        bb = g - f * BB
        src = trows_v.at[buf, :, :, pl.ds(0, 128)]
        pltpu.make_async_copy(src, out_hbm.at[f, :, bb], osem).wait()

    def drain_gather(t, buf):
        pltpu.make_async_copy(
            table2d.at[idx_v.at[pl.ds(t * 128, 128)]],
            rows_v.at[buf],
            gsem,
        ).wait()

    gather(0, 0)

    def body(t, carry):
        for buf in range(2):
            tt = t + buf
            drain_gather(tt, buf)
            nxt = tt + 1

            @pl.when(nxt < PER_W)
            def _():
                gather(nxt, 1 - buf)

            @pl.when(tt >= 2)
            def _():
                drain_out(tt - 2, buf)

            transpose(buf)
            writeout(tt, buf)
        return carry

    lax.fori_loop(0, PER_W // 2, lambda i, c: body(2 * i, c), 0, unroll=False)
    drain_out(PER_W - 2, 0)
    drain_out(PER_W - 1, 1)


def kernel(inputs, L):
    idx_t = inputs.T.reshape(-1).astype(jnp.int32)
    p5 = _gather_kernel(idx_t, L)
    return p5.transpose(2, 4, 0, 1, 3).reshape(B, F, DIM)
